# baseline (speedup 1.0000x reference)
import dataclasses
import functools

import jax
import jax.numpy as jnp
from jax import lax
from jax.experimental import pallas as pl
from jax.experimental.pallas import tpu as pltpu
from jax.experimental.pallas import tpu_sc as plsc

NN = 10000
NP = 10240
NE = 160000
DF = 128
NG = 8

NC, NS = 2, 16
NW = NC * NS
B = 128
NBLK = NE // B
KMAX = -(-NBLK // NW)

_MESH = dict(core_axis_name="c", subcore_axis_name="s")

_SC_CP = pltpu.CompilerParams()
if "needs_layout_passes" in pltpu.CompilerParams.__dataclass_fields__:
    _SC_CP = dataclasses.replace(_SC_CP, needs_layout_passes=False)


def _wid():
    return lax.axis_index("s") * NC + lax.axis_index("c")


def _sc_count(ei0, ei1):
    @functools.partial(
        pl.kernel,
        mesh=plsc.VectorSubcoreMesh(**_MESH),
        out_type=jax.ShapeDtypeStruct((NC, NP), jnp.float32),
        scratch_types=[
            pltpu.VMEM_SHARED((NP,), jnp.float32),
            pltpu.VMEM((640,), jnp.float32),
            pltpu.VMEM((B,), jnp.float32),
            pltpu.VMEM((B,), jnp.int32),
            pltpu.VMEM((B,), jnp.int32),
        ],
    )
    def k(ei0_hbm, ei1_hbm, out_hbm, cnt_spm, zbuf, ones, i0, i1):
        cid = lax.axis_index("c")
        sid = lax.axis_index("s")
        wid = _wid()

        @pl.loop(0, 640, step=16)
        def _(i):
            zbuf[pl.ds(i, 16)] = jnp.zeros((16,), jnp.float32)

        @pl.loop(0, B, step=16)
        def _(i):
            ones[pl.ds(i, 16)] = jnp.full((16,), 1.0, jnp.float32)

        pltpu.sync_copy(zbuf, cnt_spm.at[pl.ds(sid * 640, 640)])

        plsc.subcore_barrier()

        @pl.loop(0, KMAX)
        def _(k_):
            blk = wid + k_ * NW

            @pl.when(blk < NBLK)
            def _():
                base = blk * B
                pltpu.sync_copy(ei0_hbm.at[pl.ds(base, B)], i0)
                pltpu.sync_copy(ei1_hbm.at[pl.ds(base, B)], i1)
                pltpu.sync_copy(ones, cnt_spm.at[i0], add=True)
                pltpu.sync_copy(ones, cnt_spm.at[i1], add=True)

        plsc.subcore_barrier()

        pltpu.sync_copy(cnt_spm.at[pl.ds(sid * 640, 640)],
                        out_hbm.at[cid].at[pl.ds(sid * 640, 640)])

    return k(ei0, ei1)


def _sc_edge_deg(cnt, ei0, ei1):
    @functools.partial(
        pl.kernel,
        mesh=plsc.VectorSubcoreMesh(**_MESH),
        compiler_params=_SC_CP,
        out_type=jax.ShapeDtypeStruct((NE,), jnp.float32),
        scratch_types=[
            pltpu.VMEM((NP,), jnp.float32),
            pltpu.VMEM((B,), jnp.int32),
            pltpu.VMEM((B,), jnp.int32),
            pltpu.VMEM((B,), jnp.float32),
        ],
    )
    def k(cnt_hbm, ei0_hbm, ei1_hbm, d_hbm, cntv, i0, i1, dv):
        wid = _wid()
        pltpu.sync_copy(cnt_hbm, cntv)

        @pl.loop(0, KMAX)
        def _(k_):
            blk = wid + k_ * NW

            @pl.when(blk < NBLK)
            def _():
                base = blk * B
                pltpu.sync_copy(ei0_hbm.at[pl.ds(base, B)], i0)
                pltpu.sync_copy(ei1_hbm.at[pl.ds(base, B)], i1)

                @pl.loop(0, B, step=16)
                def _(j):
                    one = jnp.full((16,), 1.0, jnp.float32)
                    zero = jnp.zeros((16,), jnp.float32)
                    c0 = plsc.load_gather(cntv, [i0[pl.ds(j, 16)]])
                    c1 = plsc.load_gather(cntv, [i1[pl.ds(j, 16)]])
                    m0 = jnp.where(c0 != one, one, zero)
                    m1 = jnp.where(c1 != one, one, zero)
                    dv[pl.ds(j, 16)] = one / (one + m0 + m1)

                pltpu.sync_copy(dv, d_hbm.at[pl.ds(base, B)])

    return k(cnt, ei0, ei1)


def _sc_gather_sum(tbl, ei0, ei1):
    @functools.partial(
        pl.kernel,
        mesh=plsc.VectorSubcoreMesh(**_MESH),
        out_type=jax.ShapeDtypeStruct((NE, DF), jnp.float32),
        scratch_types=[
            pltpu.VMEM((B, DF), jnp.float32),
            pltpu.VMEM((B, DF), jnp.float32),
            pltpu.VMEM((B, DF), jnp.float32),
            pltpu.VMEM((B, DF), jnp.float32),
            pltpu.VMEM((B,), jnp.int32),
            pltpu.VMEM((B,), jnp.int32),
            pltpu.VMEM((B,), jnp.int32),
            pltpu.VMEM((B,), jnp.int32),
            pltpu.SemaphoreType.DMA,
            pltpu.SemaphoreType.DMA,
        ],
    )
    def k(t_hbm, ei0_hbm, ei1_hbm, g_hbm,
          r0a, r1a, r0b, r1b, i0a, i1a, i0b, i1b, sema, semb):
        wid = _wid()

        def issue(k_, i0, i1, r0, r1, sem):
            blk = wid + k_ * NW

            @pl.when(blk < NBLK)
            def _():
                base = blk * B
                pltpu.sync_copy(ei0_hbm.at[pl.ds(base, B)], i0)
                pltpu.sync_copy(ei1_hbm.at[pl.ds(base, B)], i1)
                pltpu.async_copy(t_hbm.at[i0], r0, sem)
                pltpu.async_copy(t_hbm.at[i1], r1, sem)

        def finish(k_, i0, i1, r0, r1, sem):
            blk = wid + k_ * NW

            @pl.when(blk < NBLK)
            def _():
                base = blk * B
                pltpu.make_async_copy(t_hbm.at[i0], r0, sem).wait()
                pltpu.make_async_copy(t_hbm.at[i1], r1, sem).wait()

                @pl.loop(0, B)
                def _(r):
                    for cc in range(0, DF, 16):
                        r0[r, pl.ds(cc, 16)] = (r0[r, pl.ds(cc, 16)]
                                                + r1[r, pl.ds(cc, 16)])

                pltpu.sync_copy(r0, g_hbm.at[pl.ds(base, B)])

        issue(0, i0a, i1a, r0a, r1a, sema)

        @pl.loop(0, KMAX, step=2)
        def _(k_):
            issue(k_ + 1, i0b, i1b, r0b, r1b, semb)
            finish(k_, i0a, i1a, r0a, r1a, sema)
            issue(k_ + 2, i0a, i1a, r0a, r1a, sema)
            finish(k_ + 1, i0b, i1b, r0b, r1b, semb)

    return k(tbl, ei0, ei1)


def _sc_scatter(vals, ei0, ei1):
    @functools.partial(
        pl.kernel,
        mesh=plsc.VectorSubcoreMesh(**_MESH),
        out_type=jax.ShapeDtypeStruct((NC, NP, DF), jnp.float32),
        scratch_types=[
            pltpu.VMEM_SHARED((NP, DF), jnp.float32),
            pltpu.VMEM((B, DF), jnp.float32),
            pltpu.VMEM((B, DF), jnp.float32),
            pltpu.VMEM((B,), jnp.int32),
            pltpu.VMEM((B,), jnp.int32),
            pltpu.VMEM((B,), jnp.int32),
            pltpu.VMEM((B,), jnp.int32),
            pltpu.SemaphoreType.DMA,
            pltpu.SemaphoreType.DMA,
        ],
    )
    def k(v_hbm, ei0_hbm, ei1_hbm, out_hbm, s_spm,
          rowsa, rowsb, i0a, i1a, i0b, i1b, sema, semb):
        cid = lax.axis_index("c")
        sid = lax.axis_index("s")
        wid = _wid()

        @pl.loop(0, B)
        def _(r):
            @pl.loop(0, DF, step=16)
            def _(cc):
                rowsa[r, pl.ds(cc, 16)] = jnp.zeros((16,), jnp.float32)

        def issue(k_, i0, i1, rows, sem):
            blk = wid + k_ * NW

            @pl.when(blk < NBLK)
            def _():
                base = blk * B
                pltpu.sync_copy(ei0_hbm.at[pl.ds(base, B)], i0)
                pltpu.sync_copy(ei1_hbm.at[pl.ds(base, B)], i1)
                pltpu.async_copy(v_hbm.at[pl.ds(base, B)], rows, sem)

        def finish(k_, i0, i1, rows, sem):
            blk = wid + k_ * NW

            @pl.when(blk < NBLK)
            def _():
                base = blk * B
                pltpu.make_async_copy(v_hbm.at[pl.ds(base, B)], rows,
                                      sem).wait()
                pltpu.sync_copy(rows, s_spm.at[i0], add=True)
                pltpu.sync_copy(rows, s_spm.at[i1], add=True)

        @pl.loop(0, 5)
        def _(z):
            pltpu.sync_copy(rowsa, s_spm.at[pl.ds(sid * 640 + z * B, B)])

        issue(0, i0a, i1a, rowsa, sema)

        plsc.subcore_barrier()

        @pl.loop(0, KMAX, step=2)
        def _(k_):
            issue(k_ + 1, i0b, i1b, rowsb, semb)
            finish(k_, i0a, i1a, rowsa, sema)
            issue(k_ + 2, i0a, i1a, rowsa, sema)
            finish(k_ + 1, i0b, i1b, rowsb, semb)

        plsc.subcore_barrier()

        @pl.loop(0, 5)
        def _(z):
            off = sid * 640 + z * B
            pltpu.sync_copy(s_spm.at[pl.ds(off, B)],
                            out_hbm.at[cid].at[pl.ds(off, B)])

    return k(vals, ei0, ei1)


_HI = jax.lax.Precision.DEFAULT
_R = 4000
_R2 = 1024


def _tc_pre(cnt2):
    def body(c_ref, cnt_ref, inv_ref):
        c = c_ref[0] + c_ref[1]
        cnt_ref[...] = c
        ge2 = c >= 2.0
        inv_ref[...] = jnp.where(ge2, 1.0 / jnp.where(ge2, c, 1.0), 0.0)

    return pl.pallas_call(
        body,
        in_specs=[pl.BlockSpec((NC, NP), lambda: (0, 0))],
        out_specs=[pl.BlockSpec((NP,), lambda: (0,)),
                   pl.BlockSpec((NP,), lambda: (0,))],
        out_shape=[jax.ShapeDtypeStruct((NP,), jnp.float32),
                   jax.ShapeDtypeStruct((NP,), jnp.float32)],
    )(cnt2)


def _tc_scale(g):
    def body(g_ref, ea_ref):
        ea_ref[...] = g_ref[...] * 0.5

    return pl.pallas_call(
        body,
        grid=(NE // _R,),
        in_specs=[pl.BlockSpec((_R, DF), lambda i: (i, 0))],
        out_specs=[pl.BlockSpec((_R, DF), lambda i: (i, 0))],
        out_shape=[jax.ShapeDtypeStruct((NE, DF), jnp.float32)],
    )(g)[0]


def _tc_xh(ea, w, wm=None):
    if wm is None:
        def body(ea_ref, w_ref, xh_ref):
            xh_ref[...] = jnp.dot(ea_ref[...], w_ref[...],
                                  preferred_element_type=jnp.float32,
                                  precision=_HI)

        return pl.pallas_call(
            body,
            grid=(NE // _R,),
            in_specs=[pl.BlockSpec((_R, DF), lambda i: (i, 0)),
                      pl.BlockSpec((DF, DF), lambda i: (0, 0))],
            out_specs=[pl.BlockSpec((_R, DF), lambda i: (i, 0))],
            out_shape=[jax.ShapeDtypeStruct((NE, DF), jnp.float32)],
        )(ea, w)[0]

    def body(ea_ref, w_ref, wm_ref, xh_ref, p_ref):
        ea_ = ea_ref[...]
        xh_ref[...] = jnp.dot(ea_, w_ref[...],
                              preferred_element_type=jnp.float32, precision=_HI)
        p_ref[...] = jnp.dot(ea_, wm_ref[...],
                             preferred_element_type=jnp.float32, precision=_HI)

    return pl.pallas_call(
        body,
        grid=(NE // _R,),
        in_specs=[pl.BlockSpec((_R, DF), lambda i: (i, 0)),
                  pl.BlockSpec((DF, DF), lambda i: (0, 0)),
                  pl.BlockSpec((DF, 1), lambda i: (0, 0))],
        out_specs=[pl.BlockSpec((_R, DF), lambda i: (i, 0)),
                   pl.BlockSpec((_R, 1), lambda i: (i, 0))],
        out_shape=[jax.ShapeDtypeStruct((NE, DF), jnp.float32),
                   jax.ShapeDtypeStruct((NE, 1), jnp.float32)],
    )(ea, w, wm)


def _tc_et(s2, inv1, w):
    def body(s_ref, inv_ref, w_ref, et_ref):
        sn = (s_ref[0] + s_ref[1]) * inv_ref[...]
        et_ref[...] = jnp.dot(sn, w_ref[...],
                              preferred_element_type=jnp.float32, precision=_HI)

    return pl.pallas_call(
        body,
        grid=(NP // _R2,),
        in_specs=[pl.BlockSpec((NC, _R2, DF), lambda i: (0, i, 0)),
                  pl.BlockSpec((_R2, 1), lambda i: (i, 0)),
                  pl.BlockSpec((DF, DF), lambda i: (0, 0))],
        out_specs=[pl.BlockSpec((_R2, DF), lambda i: (i, 0))],
        out_shape=[jax.ShapeDtypeStruct((NP, DF), jnp.float32)],
    )(s2, inv1, w)[0]


def _tc_ea(g, xh, d2, brow):
    def body(g_ref, xh_ref, d_ref, b_ref, ea_ref):
        v = (g_ref[...] + xh_ref[...]) * d_ref[...] + b_ref[...]
        ea_ref[...] = jnp.maximum(v, 0.0)

    return pl.pallas_call(
        body,
        grid=(NE // _R,),
        in_specs=[pl.BlockSpec((_R, DF), lambda i: (i, 0)),
                  pl.BlockSpec((_R, DF), lambda i: (i, 0)),
                  pl.BlockSpec((_R, 1), lambda i: (i, 0)),
                  pl.BlockSpec((1, DF), lambda i: (0, 0))],
        out_specs=[pl.BlockSpec((_R, DF), lambda i: (i, 0))],
        out_shape=[jax.ShapeDtypeStruct((NE, DF), jnp.float32)],
    )(g, xh, d2, brow)[0]


def _tc_last(g, xh, d2, brow, wm):
    def body(g_ref, xh_ref, d_ref, b_ref, wm_ref, p_ref):
        v = (g_ref[...] + xh_ref[...]) * d_ref[...] + b_ref[...]
        ea = jnp.maximum(v, 0.0)
        p_ref[...] = jnp.dot(ea, wm_ref[...],
                             preferred_element_type=jnp.float32, precision=_HI)

    return pl.pallas_call(
        body,
        grid=(NE // _R,),
        in_specs=[pl.BlockSpec((_R, DF), lambda i: (i, 0)),
                  pl.BlockSpec((_R, DF), lambda i: (i, 0)),
                  pl.BlockSpec((_R, 1), lambda i: (i, 0)),
                  pl.BlockSpec((1, DF), lambda i: (0, 0)),
                  pl.BlockSpec((DF, 1), lambda i: (0, 0))],
        out_specs=[pl.BlockSpec((_R, 1), lambda i: (i, 0))],
        out_shape=[jax.ShapeDtypeStruct((NE, 1), jnp.float32)],
    )(g, xh, d2, brow, wm)[0]


def _tc_soft(p0, p1, p2, bm, batch, eit):
    def body(p0_ref, p1_ref, p2_ref, bm_ref, b_ref, e_ref, o_ref):
        p = p0_ref[...] + p1_ref[...] + p2_ref[...] + bm_ref[0, 0]
        bt = b_ref[...]
        ei = e_ref[...]
        eb = jnp.zeros(ei.shape, jnp.int32)
        for g in range(1, NG):
            t_g = jnp.sum((bt < g).astype(jnp.int32))
            eb = eb + (ei >= t_g).astype(jnp.int32)
        msel = jnp.zeros(p.shape, jnp.float32)
        for g in range(NG):
            mg = jnp.max(jnp.where(eb == g, p, -jnp.inf))
            mg = jnp.where(jnp.isfinite(mg), mg, 0.0)
            msel = jnp.where(eb == g, mg, msel)
        ex = jnp.exp(p - msel)
        dsel = jnp.zeros(p.shape, jnp.float32)
        for g in range(NG):
            sg = jnp.sum(jnp.where(eb == g, ex, 0.0))
            dsel = jnp.where(eb == g, sg, dsel)
        o_ref[...] = ex / (dsel + 1e-16)

    nr = NE // DF
    return pl.pallas_call(
        body,
        in_specs=[pl.BlockSpec((nr, DF), lambda: (0, 0)),
                  pl.BlockSpec((nr, DF), lambda: (0, 0)),
                  pl.BlockSpec((nr, DF), lambda: (0, 0)),
                  pl.BlockSpec((1, 1), lambda: (0, 0)),
                  pl.BlockSpec((NN,), lambda: (0,)),
                  pl.BlockSpec((nr, DF), lambda: (0, 0))],
        out_specs=[pl.BlockSpec((nr, DF), lambda: (0, 0))],
        out_shape=[jax.ShapeDtypeStruct((nr, DF), jnp.float32)],
    )(p0, p1, p2, bm, batch, eit)[0]


def kernel(x, edge_index, edge_attr, batch, W0, b0, W1, b1, W2, b2,
           W_mlp, b_mlp):
    del edge_attr
    ei0 = edge_index[0]
    ei1 = edge_index[1]

    cnt2 = _sc_count(ei0, ei1)
    cnt, inv = _tc_pre(cnt2)
    inv1 = inv.reshape(NP, 1)
    d2 = _sc_edge_deg(cnt, ei0, ei1).reshape(NE, 1)

    x_dep = lax.optimization_barrier((x, d2))[0]
    g = _sc_gather_sum(x_dep, ei0, ei1)
    ea = _tc_scale(g)

    ws = (W0, W1, W2)
    bs = (b0.reshape(1, DF), b1.reshape(1, DF), b2.reshape(1, DF))
    wms = (W_mlp[0:DF], W_mlp[DF:2 * DF], W_mlp[2 * DF:3 * DF])

    ps = []
    for li in range(3):
        s2 = _sc_scatter(ea, ei0, ei1)
        if li == 0:
            xh = _tc_xh(ea, ws[0])
        else:
            xh, p_prev = _tc_xh(ea, ws[li], wms[li - 1])
            ps.append(p_prev.reshape(NE // DF, DF))
        et = _tc_et(s2, inv1, ws[li])
        g = _sc_gather_sum(et, ei0, ei1)
        if li < 2:
            ea = _tc_ea(g, xh, d2, bs[li])
        else:
            p_l = _tc_last(g, xh, d2, bs[li], wms[li])
            ps.append(p_l.reshape(NE // DF, DF))

    out = _tc_soft(ps[0], ps[1], ps[2], b_mlp.reshape(1, 1), batch,
                   ei0.reshape(NE // DF, DF))
    return out.reshape(NE, 1)


# speedup vs baseline: 1.0519x; 1.0519x over previous
import dataclasses
import functools

import jax
import jax.numpy as jnp
from jax import lax
from jax.experimental import pallas as pl
from jax.experimental.pallas import tpu as pltpu
from jax.experimental.pallas import tpu_sc as plsc

NN = 10000
NP = 10240
NE = 160000
DF = 128
NG = 8

NC, NS = 2, 16
NW = NC * NS
B = 128
NBLK = NE // B
KMAX = -(-NBLK // NW)

_MESH = dict(core_axis_name="c", subcore_axis_name="s")

_SC_CP = pltpu.CompilerParams()
if "needs_layout_passes" in pltpu.CompilerParams.__dataclass_fields__:
    _SC_CP = dataclasses.replace(_SC_CP, needs_layout_passes=False)


def _wid():
    return lax.axis_index("s") * NC + lax.axis_index("c")


def _sc_count(ei0, ei1):
    @functools.partial(
        pl.kernel,
        mesh=plsc.VectorSubcoreMesh(**_MESH),
        out_type=jax.ShapeDtypeStruct((NC, NP), jnp.float32),
        scratch_types=[
            pltpu.VMEM_SHARED((NP,), jnp.float32),
            pltpu.VMEM((640,), jnp.float32),
            pltpu.VMEM((B,), jnp.float32),
            pltpu.VMEM((B,), jnp.int32),
            pltpu.VMEM((B,), jnp.int32),
        ],
    )
    def k(ei0_hbm, ei1_hbm, out_hbm, cnt_spm, zbuf, ones, i0, i1):
        cid = lax.axis_index("c")
        sid = lax.axis_index("s")
        wid = _wid()

        @pl.loop(0, 640, step=16)
        def _(i):
            zbuf[pl.ds(i, 16)] = jnp.zeros((16,), jnp.float32)

        @pl.loop(0, B, step=16)
        def _(i):
            ones[pl.ds(i, 16)] = jnp.full((16,), 1.0, jnp.float32)

        pltpu.sync_copy(zbuf, cnt_spm.at[pl.ds(sid * 640, 640)])

        plsc.subcore_barrier()

        @pl.loop(0, KMAX)
        def _(k_):
            blk = wid + k_ * NW

            @pl.when(blk < NBLK)
            def _():
                base = blk * B
                pltpu.sync_copy(ei0_hbm.at[pl.ds(base, B)], i0)
                pltpu.sync_copy(ei1_hbm.at[pl.ds(base, B)], i1)
                pltpu.sync_copy(ones, cnt_spm.at[i0], add=True)
                pltpu.sync_copy(ones, cnt_spm.at[i1], add=True)

        plsc.subcore_barrier()

        pltpu.sync_copy(cnt_spm.at[pl.ds(sid * 640, 640)],
                        out_hbm.at[cid].at[pl.ds(sid * 640, 640)])

    return k(ei0, ei1)


def _sc_edge_deg(cnt, ei0, ei1):
    @functools.partial(
        pl.kernel,
        mesh=plsc.VectorSubcoreMesh(**_MESH),
        compiler_params=_SC_CP,
        out_type=jax.ShapeDtypeStruct((NE,), jnp.float32),
        scratch_types=[
            pltpu.VMEM((NP,), jnp.float32),
            pltpu.VMEM((B,), jnp.int32),
            pltpu.VMEM((B,), jnp.int32),
            pltpu.VMEM((B,), jnp.float32),
        ],
    )
    def k(cnt_hbm, ei0_hbm, ei1_hbm, d_hbm, cntv, i0, i1, dv):
        wid = _wid()
        pltpu.sync_copy(cnt_hbm, cntv)

        @pl.loop(0, KMAX)
        def _(k_):
            blk = wid + k_ * NW

            @pl.when(blk < NBLK)
            def _():
                base = blk * B
                pltpu.sync_copy(ei0_hbm.at[pl.ds(base, B)], i0)
                pltpu.sync_copy(ei1_hbm.at[pl.ds(base, B)], i1)

                @pl.loop(0, B, step=16)
                def _(j):
                    one = jnp.full((16,), 1.0, jnp.float32)
                    zero = jnp.zeros((16,), jnp.float32)
                    c0 = plsc.load_gather(cntv, [i0[pl.ds(j, 16)]])
                    c1 = plsc.load_gather(cntv, [i1[pl.ds(j, 16)]])
                    m0 = jnp.where(c0 != one, one, zero)
                    m1 = jnp.where(c1 != one, one, zero)
                    dv[pl.ds(j, 16)] = one / (one + m0 + m1)

                pltpu.sync_copy(dv, d_hbm.at[pl.ds(base, B)])

    return k(cnt, ei0, ei1)


def _sc_gather_fused(tbl, ei0, ei1, mode, xh=None, d=None, bias=None, wm=None):
    has_x = mode in ("ea", "last")
    out_ty = (jax.ShapeDtypeStruct((NBLK, B), jnp.float32) if mode == "last"
              else jax.ShapeDtypeStruct((NE, DF), jnp.float32))
    scratch = [
        pltpu.VMEM((B, DF), jnp.float32),
        pltpu.VMEM((B, DF), jnp.float32),
        pltpu.VMEM((B, DF), jnp.float32),
        pltpu.VMEM((B, DF), jnp.float32),
        pltpu.VMEM((B,), jnp.int32),
        pltpu.VMEM((B,), jnp.int32),
        pltpu.VMEM((B,), jnp.int32),
        pltpu.VMEM((B,), jnp.int32),
        pltpu.SemaphoreType.DMA,
        pltpu.SemaphoreType.DMA,
    ]
    if has_x:
        scratch += [
            pltpu.VMEM((B, DF), jnp.float32),
            pltpu.VMEM((B, DF), jnp.float32),
            pltpu.VMEM((B,), jnp.float32),
            pltpu.VMEM((B,), jnp.float32),
            pltpu.VMEM((DF,), jnp.float32),
        ]
    if mode == "last":
        scratch += [
            pltpu.VMEM((DF,), jnp.float32),
            pltpu.VMEM((B,), jnp.float32),
        ]

    def body(refs):
        if mode == "scale":
            (t_hbm, ei0_hbm, ei1_hbm, g_hbm,
             r0a, r1a, r0b, r1b, i0a, i1a, i0b, i1b, sema, semb) = refs
        elif mode == "ea":
            (t_hbm, ei0_hbm, ei1_hbm, xh_hbm, d_hbm, b_hbm, g_hbm,
             r0a, r1a, r0b, r1b, i0a, i1a, i0b, i1b, sema, semb,
             xha, xhb, da, db, bbuf) = refs
        else:
            (t_hbm, ei0_hbm, ei1_hbm, xh_hbm, d_hbm, b_hbm, wm_hbm, g_hbm,
             r0a, r1a, r0b, r1b, i0a, i1a, i0b, i1b, sema, semb,
             xha, xhb, da, db, bbuf, wmbuf, pbuf) = refs
        wid = _wid()
        if has_x:
            pltpu.sync_copy(b_hbm, bbuf)
        if mode == "last":
            pltpu.sync_copy(wm_hbm, wmbuf)

        def issue(k_, i0, i1, r0, r1, xh_b, d_b, sem):
            blk = wid + k_ * NW

            @pl.when(blk < NBLK)
            def _():
                base = blk * B
                pltpu.sync_copy(ei0_hbm.at[pl.ds(base, B)], i0)
                pltpu.sync_copy(ei1_hbm.at[pl.ds(base, B)], i1)
                if has_x:
                    pltpu.sync_copy(d_hbm.at[pl.ds(base, B)], d_b)
                    pltpu.async_copy(xh_hbm.at[pl.ds(base, B)], xh_b, sem)
                pltpu.async_copy(t_hbm.at[i0], r0, sem)
                pltpu.async_copy(t_hbm.at[i1], r1, sem)

        def finish(k_, i0, i1, r0, r1, xh_b, d_b, sem):
            blk = wid + k_ * NW

            @pl.when(blk < NBLK)
            def _():
                base = blk * B
                pltpu.make_async_copy(t_hbm.at[i0], r0, sem).wait()
                pltpu.make_async_copy(t_hbm.at[i1], r1, sem).wait()
                if has_x:
                    pltpu.make_async_copy(xh_hbm.at[pl.ds(base, B)], xh_b,
                                          sem).wait()

                if mode == "scale":
                    @pl.loop(0, B)
                    def _(r):
                        for cc in range(0, DF, 16):
                            c = pl.ds(cc, 16)
                            r0[r, c] = (r0[r, c] + r1[r, c]) * 0.5

                    pltpu.sync_copy(r0, g_hbm.at[pl.ds(base, B)])
                elif mode == "ea":
                    @pl.loop(0, B, step=16)
                    def _(rr):
                        dvec = d_b[pl.ds(rr, 16)]
                        for j in range(16):
                            r = rr + j
                            dv = jnp.full((16,), dvec[j], jnp.float32)
                            for cc in range(0, DF, 16):
                                c = pl.ds(cc, 16)
                                v = (r0[r, c] + r1[r, c] + xh_b[r, c]) * dv
                                r0[r, c] = jnp.maximum(v + bbuf[c], 0.0)

                    pltpu.sync_copy(r0, g_hbm.at[pl.ds(base, B)])
                else:
                    lane = lax.iota(jnp.int32, 16)

                    @pl.loop(0, B, step=16)
                    def _(rr):
                        dvec = d_b[pl.ds(rr, 16)]
                        pvec = jnp.zeros((16,), jnp.float32)
                        for j in range(16):
                            r = rr + j
                            dv = jnp.full((16,), dvec[j], jnp.float32)
                            acc = jnp.zeros((16,), jnp.float32)
                            for cc in range(0, DF, 16):
                                c = pl.ds(cc, 16)
                                v = (r0[r, c] + r1[r, c] + xh_b[r, c]) * dv
                                ea_c = jnp.maximum(v + bbuf[c], 0.0)
                                acc = acc + ea_c * wmbuf[c]
                            s_j = jnp.full((16,), jnp.sum(acc), jnp.float32)
                            pvec = jnp.where(lane == j, s_j, pvec)
                        pbuf[pl.ds(rr, 16)] = pvec

                    pltpu.sync_copy(pbuf, g_hbm.at[blk])

        issue(0, i0a, i1a, r0a, r1a, xha if has_x else None,
              da if has_x else None, sema)

        @pl.loop(0, KMAX, step=2)
        def _(k_):
            issue(k_ + 1, i0b, i1b, r0b, r1b, xhb if has_x else None,
                  db if has_x else None, semb)
            finish(k_, i0a, i1a, r0a, r1a, xha if has_x else None,
                   da if has_x else None, sema)
            issue(k_ + 2, i0a, i1a, r0a, r1a, xha if has_x else None,
                  da if has_x else None, sema)
            finish(k_ + 1, i0b, i1b, r0b, r1b, xhb if has_x else None,
                   db if has_x else None, semb)

    @functools.partial(
        pl.kernel,
        mesh=plsc.VectorSubcoreMesh(**_MESH),
        compiler_params=_SC_CP,
        out_type=out_ty,
        scratch_types=scratch,
    )
    def k(*refs):
        body(refs)

    if mode == "scale":
        return k(tbl, ei0, ei1)
    if mode == "ea":
        return k(tbl, ei0, ei1, xh, d, bias)
    return k(tbl, ei0, ei1, xh, d, bias, wm)


def _sc_scatter(vals, ei0, ei1):
    @functools.partial(
        pl.kernel,
        mesh=plsc.VectorSubcoreMesh(**_MESH),
        out_type=jax.ShapeDtypeStruct((NC, NP, DF), jnp.float32),
        scratch_types=[
            pltpu.VMEM_SHARED((NP, DF), jnp.float32),
            pltpu.VMEM((B, DF), jnp.float32),
            pltpu.VMEM((B, DF), jnp.float32),
            pltpu.VMEM((B,), jnp.int32),
            pltpu.VMEM((B,), jnp.int32),
            pltpu.VMEM((B,), jnp.int32),
            pltpu.VMEM((B,), jnp.int32),
            pltpu.SemaphoreType.DMA,
            pltpu.SemaphoreType.DMA,
        ],
    )
    def k(v_hbm, ei0_hbm, ei1_hbm, out_hbm, s_spm,
          rowsa, rowsb, i0a, i1a, i0b, i1b, sema, semb):
        cid = lax.axis_index("c")
        sid = lax.axis_index("s")
        wid = _wid()

        @pl.loop(0, B)
        def _(r):
            @pl.loop(0, DF, step=16)
            def _(cc):
                rowsa[r, pl.ds(cc, 16)] = jnp.zeros((16,), jnp.float32)

        def issue(k_, i0, i1, rows, sem):
            blk = wid + k_ * NW

            @pl.when(blk < NBLK)
            def _():
                base = blk * B
                pltpu.sync_copy(ei0_hbm.at[pl.ds(base, B)], i0)
                pltpu.sync_copy(ei1_hbm.at[pl.ds(base, B)], i1)
                pltpu.async_copy(v_hbm.at[pl.ds(base, B)], rows, sem)

        def finish(k_, i0, i1, rows, sem):
            blk = wid + k_ * NW

            @pl.when(blk < NBLK)
            def _():
                base = blk * B
                pltpu.make_async_copy(v_hbm.at[pl.ds(base, B)], rows,
                                      sem).wait()
                pltpu.sync_copy(rows, s_spm.at[i0], add=True)
                pltpu.sync_copy(rows, s_spm.at[i1], add=True)

        @pl.loop(0, 5)
        def _(z):
            pltpu.sync_copy(rowsa, s_spm.at[pl.ds(sid * 640 + z * B, B)])

        issue(0, i0a, i1a, rowsa, sema)

        plsc.subcore_barrier()

        @pl.loop(0, KMAX, step=2)
        def _(k_):
            issue(k_ + 1, i0b, i1b, rowsb, semb)
            finish(k_, i0a, i1a, rowsa, sema)
            issue(k_ + 2, i0a, i1a, rowsa, sema)
            finish(k_ + 1, i0b, i1b, rowsb, semb)

        plsc.subcore_barrier()

        @pl.loop(0, 5)
        def _(z):
            off = sid * 640 + z * B
            pltpu.sync_copy(s_spm.at[pl.ds(off, B)],
                            out_hbm.at[cid].at[pl.ds(off, B)])

    return k(vals, ei0, ei1)


_HI = jax.lax.Precision.DEFAULT
_R = 4000
_R2 = 1024


def _tc_pre(cnt2):
    def body(c_ref, cnt_ref, inv_ref):
        c = c_ref[0] + c_ref[1]
        cnt_ref[...] = c
        ge2 = c >= 2.0
        inv_ref[...] = jnp.where(ge2, 1.0 / jnp.where(ge2, c, 1.0), 0.0)

    return pl.pallas_call(
        body,
        in_specs=[pl.BlockSpec((NC, NP), lambda: (0, 0))],
        out_specs=[pl.BlockSpec((NP,), lambda: (0,)),
                   pl.BlockSpec((NP,), lambda: (0,))],
        out_shape=[jax.ShapeDtypeStruct((NP,), jnp.float32),
                   jax.ShapeDtypeStruct((NP,), jnp.float32)],
    )(cnt2)


def _tc_xh(ea, w, wm=None):
    if wm is None:
        def body(ea_ref, w_ref, xh_ref):
            xh_ref[...] = jnp.dot(ea_ref[...], w_ref[...],
                                  preferred_element_type=jnp.float32,
                                  precision=_HI)

        return pl.pallas_call(
            body,
            grid=(NE // _R,),
            in_specs=[pl.BlockSpec((_R, DF), lambda i: (i, 0)),
                      pl.BlockSpec((DF, DF), lambda i: (0, 0))],
            out_specs=[pl.BlockSpec((_R, DF), lambda i: (i, 0))],
            out_shape=[jax.ShapeDtypeStruct((NE, DF), jnp.float32)],
        )(ea, w)[0]

    def body(ea_ref, w_ref, wm_ref, xh_ref, p_ref):
        ea_ = ea_ref[...]
        xh_ref[...] = jnp.dot(ea_, w_ref[...],
                              preferred_element_type=jnp.float32, precision=_HI)
        p_ref[...] = jnp.dot(ea_, wm_ref[...],
                             preferred_element_type=jnp.float32, precision=_HI)

    return pl.pallas_call(
        body,
        grid=(NE // _R,),
        in_specs=[pl.BlockSpec((_R, DF), lambda i: (i, 0)),
                  pl.BlockSpec((DF, DF), lambda i: (0, 0)),
                  pl.BlockSpec((DF, 1), lambda i: (0, 0))],
        out_specs=[pl.BlockSpec((_R, DF), lambda i: (i, 0)),
                   pl.BlockSpec((_R, 1), lambda i: (i, 0))],
        out_shape=[jax.ShapeDtypeStruct((NE, DF), jnp.float32),
                   jax.ShapeDtypeStruct((NE, 1), jnp.float32)],
    )(ea, w, wm)


def _tc_et(s2, inv1, w):
    def body(s_ref, inv_ref, w_ref, et_ref):
        sn = (s_ref[0] + s_ref[1]) * inv_ref[...]
        et_ref[...] = jnp.dot(sn, w_ref[...],
                              preferred_element_type=jnp.float32, precision=_HI)

    return pl.pallas_call(
        body,
        grid=(NP // _R2,),
        in_specs=[pl.BlockSpec((NC, _R2, DF), lambda i: (0, i, 0)),
                  pl.BlockSpec((_R2, 1), lambda i: (i, 0)),
                  pl.BlockSpec((DF, DF), lambda i: (0, 0))],
        out_specs=[pl.BlockSpec((_R2, DF), lambda i: (i, 0))],
        out_shape=[jax.ShapeDtypeStruct((NP, DF), jnp.float32)],
    )(s2, inv1, w)[0]


def _tc_soft(p0, p1, p2, bm, batch, eit):
    def body(p0_ref, p1_ref, p2_ref, bm_ref, b_ref, e_ref, o_ref):
        p = p0_ref[...] + p1_ref[...] + p2_ref[...] + bm_ref[0, 0]
        bt = b_ref[...]
        ei = e_ref[...]
        eb = jnp.zeros(ei.shape, jnp.int32)
        for g in range(1, NG):
            t_g = jnp.sum((bt < g).astype(jnp.int32))
            eb = eb + (ei >= t_g).astype(jnp.int32)
        msel = jnp.zeros(p.shape, jnp.float32)
        for g in range(NG):
            mg = jnp.max(jnp.where(eb == g, p, -jnp.inf))
            mg = jnp.where(jnp.isfinite(mg), mg, 0.0)
            msel = jnp.where(eb == g, mg, msel)
        ex = jnp.exp(p - msel)
        dsel = jnp.zeros(p.shape, jnp.float32)
        for g in range(NG):
            sg = jnp.sum(jnp.where(eb == g, ex, 0.0))
            dsel = jnp.where(eb == g, sg, dsel)
        o_ref[...] = ex / (dsel + 1e-16)

    nr = NE // DF
    return pl.pallas_call(
        body,
        in_specs=[pl.BlockSpec((nr, DF), lambda: (0, 0)),
                  pl.BlockSpec((nr, DF), lambda: (0, 0)),
                  pl.BlockSpec((nr, DF), lambda: (0, 0)),
                  pl.BlockSpec((1, 1), lambda: (0, 0)),
                  pl.BlockSpec((NN,), lambda: (0,)),
                  pl.BlockSpec((nr, DF), lambda: (0, 0))],
        out_specs=[pl.BlockSpec((nr, DF), lambda: (0, 0))],
        out_shape=[jax.ShapeDtypeStruct((nr, DF), jnp.float32)],
    )(p0, p1, p2, bm, batch, eit)[0]


def kernel(x, edge_index, edge_attr, batch, W0, b0, W1, b1, W2, b2,
           W_mlp, b_mlp):
    del edge_attr
    ei0 = edge_index[0]
    ei1 = edge_index[1]

    cnt2 = _sc_count(ei0, ei1)
    cnt, inv = _tc_pre(cnt2)
    inv1 = inv.reshape(NP, 1)
    d1 = _sc_edge_deg(cnt, ei0, ei1)

    x_dep = lax.optimization_barrier((x, d1))[0]
    ea = _sc_gather_fused(x_dep, ei0, ei1, "scale")

    ws = (W0, W1, W2)
    bs = (b0, b1, b2)
    wms = (W_mlp[0:DF, 0], W_mlp[DF:2 * DF, 0], W_mlp[2 * DF:3 * DF, 0])

    ps = []
    for li in range(3):
        s2 = _sc_scatter(ea, ei0, ei1)
        if li == 0:
            xh = _tc_xh(ea, ws[0])
        else:
            xh, p_prev = _tc_xh(ea, ws[li], wms[li - 1].reshape(DF, 1))
            ps.append(p_prev.reshape(NE // DF, DF))
        s2b = lax.optimization_barrier((s2, xh))[0]
        et = _tc_et(s2b, inv1, ws[li])
        if li < 2:
            ea = _sc_gather_fused(et, ei0, ei1, "ea",
                                  xh=xh, d=d1, bias=bs[li])
        else:
            p_l = _sc_gather_fused(et, ei0, ei1, "last",
                                   xh=xh, d=d1, bias=bs[li], wm=wms[li])
            ps.append(p_l)

    out = _tc_soft(ps[0], ps[1], ps[2], b_mlp.reshape(1, 1), batch,
                   ei0.reshape(NE // DF, DF))
    return out.reshape(NE, 1)


# speedup vs baseline: 1.1920x; 1.1332x over previous
import dataclasses
import functools

import jax
import jax.numpy as jnp
from jax import lax
from jax.experimental import pallas as pl
from jax.experimental.pallas import tpu as pltpu
from jax.experimental.pallas import tpu_sc as plsc

NN = 10000
NP = 10240
NE = 160000
DF = 128
NG = 8

NC, NS = 2, 16
NW = NC * NS
B = 128
NBLK = NE // B
KMAX = -(-NBLK // NW)

_MESH = dict(core_axis_name="c", subcore_axis_name="s")

_SC_CP = pltpu.CompilerParams()
if "needs_layout_passes" in pltpu.CompilerParams.__dataclass_fields__:
    _SC_CP = dataclasses.replace(_SC_CP, needs_layout_passes=False)


def _wid():
    return lax.axis_index("s") * NC + lax.axis_index("c")


def _sc_count(ei0, ei1):
    @functools.partial(
        pl.kernel,
        mesh=plsc.VectorSubcoreMesh(**_MESH),
        out_type=jax.ShapeDtypeStruct((NC, NP), jnp.float32),
        scratch_types=[
            pltpu.VMEM_SHARED((NP,), jnp.float32),
            pltpu.VMEM((640,), jnp.float32),
            pltpu.VMEM((B,), jnp.float32),
            pltpu.VMEM((B,), jnp.int32),
            pltpu.VMEM((B,), jnp.int32),
        ],
    )
    def k(ei0_hbm, ei1_hbm, out_hbm, cnt_spm, zbuf, ones, i0, i1):
        cid = lax.axis_index("c")
        sid = lax.axis_index("s")
        wid = _wid()

        @pl.loop(0, 640, step=16)
        def _(i):
            zbuf[pl.ds(i, 16)] = jnp.zeros((16,), jnp.float32)

        @pl.loop(0, B, step=16)
        def _(i):
            ones[pl.ds(i, 16)] = jnp.full((16,), 1.0, jnp.float32)

        pltpu.sync_copy(zbuf, cnt_spm.at[pl.ds(sid * 640, 640)])

        plsc.subcore_barrier()

        @pl.loop(0, KMAX)
        def _(k_):
            blk = wid + k_ * NW

            @pl.when(blk < NBLK)
            def _():
                base = blk * B
                pltpu.sync_copy(ei0_hbm.at[pl.ds(base, B)], i0)
                pltpu.sync_copy(ei1_hbm.at[pl.ds(base, B)], i1)
                pltpu.sync_copy(ones, cnt_spm.at[i0], add=True)
                pltpu.sync_copy(ones, cnt_spm.at[i1], add=True)

        plsc.subcore_barrier()

        pltpu.sync_copy(cnt_spm.at[pl.ds(sid * 640, 640)],
                        out_hbm.at[cid].at[pl.ds(sid * 640, 640)])

    return k(ei0, ei1)


def _sc_edge_deg(cnt, ei0, ei1):
    @functools.partial(
        pl.kernel,
        mesh=plsc.VectorSubcoreMesh(**_MESH),
        compiler_params=_SC_CP,
        out_type=jax.ShapeDtypeStruct((NE,), jnp.float32),
        scratch_types=[
            pltpu.VMEM((NP,), jnp.float32),
            pltpu.VMEM((B,), jnp.int32),
            pltpu.VMEM((B,), jnp.int32),
            pltpu.VMEM((B,), jnp.float32),
        ],
    )
    def k(cnt_hbm, ei0_hbm, ei1_hbm, d_hbm, cntv, i0, i1, dv):
        wid = _wid()
        pltpu.sync_copy(cnt_hbm, cntv)

        @pl.loop(0, KMAX)
        def _(k_):
            blk = wid + k_ * NW

            @pl.when(blk < NBLK)
            def _():
                base = blk * B
                pltpu.sync_copy(ei0_hbm.at[pl.ds(base, B)], i0)
                pltpu.sync_copy(ei1_hbm.at[pl.ds(base, B)], i1)

                @pl.loop(0, B, step=16)
                def _(j):
                    one = jnp.full((16,), 1.0, jnp.float32)
                    zero = jnp.zeros((16,), jnp.float32)
                    c0 = plsc.load_gather(cntv, [i0[pl.ds(j, 16)]])
                    c1 = plsc.load_gather(cntv, [i1[pl.ds(j, 16)]])
                    m0 = jnp.where(c0 != one, one, zero)
                    m1 = jnp.where(c1 != one, one, zero)
                    dv[pl.ds(j, 16)] = one / (one + m0 + m1)

                pltpu.sync_copy(dv, d_hbm.at[pl.ds(base, B)])

    return k(cnt, ei0, ei1)


def _sc_gather_fused(tbl, ei0, ei1, mode, xh=None, d=None, bias=None, wm=None):
    has_x = mode in ("ea", "last")
    out_ty = (jax.ShapeDtypeStruct((NBLK, B), jnp.float32) if mode == "last"
              else jax.ShapeDtypeStruct((NE, DF), jnp.float32))
    scratch = [
        pltpu.VMEM((B, DF), jnp.float32),
        pltpu.VMEM((B, DF), jnp.float32),
        pltpu.VMEM((B, DF), jnp.float32),
        pltpu.VMEM((B, DF), jnp.float32),
        pltpu.VMEM((B,), jnp.int32),
        pltpu.VMEM((B,), jnp.int32),
        pltpu.VMEM((B,), jnp.int32),
        pltpu.VMEM((B,), jnp.int32),
        pltpu.SemaphoreType.DMA,
        pltpu.SemaphoreType.DMA,
    ]
    if has_x:
        scratch += [
            pltpu.VMEM((B, DF), jnp.float32),
            pltpu.VMEM((B, DF), jnp.float32),
            pltpu.VMEM((B,), jnp.float32),
            pltpu.VMEM((B,), jnp.float32),
            pltpu.VMEM((DF,), jnp.float32),
        ]
    if mode == "last":
        scratch += [
            pltpu.VMEM((DF,), jnp.float32),
            pltpu.VMEM((B,), jnp.float32),
        ]

    def body(refs):
        if mode in ("scale", "sum"):
            (t_hbm, ei0_hbm, ei1_hbm, g_hbm,
             r0a, r1a, r0b, r1b, i0a, i1a, i0b, i1b, sema, semb) = refs
        elif mode == "ea":
            (t_hbm, ei0_hbm, ei1_hbm, xh_hbm, d_hbm, b_hbm, g_hbm,
             r0a, r1a, r0b, r1b, i0a, i1a, i0b, i1b, sema, semb,
             xha, xhb, da, db, bbuf) = refs
        else:
            (t_hbm, ei0_hbm, ei1_hbm, xh_hbm, d_hbm, b_hbm, wm_hbm, g_hbm,
             r0a, r1a, r0b, r1b, i0a, i1a, i0b, i1b, sema, semb,
             xha, xhb, da, db, bbuf, wmbuf, pbuf) = refs
        wid = _wid()
        if has_x:
            pltpu.sync_copy(b_hbm, bbuf)
        if mode == "last":
            pltpu.sync_copy(wm_hbm, wmbuf)

        def issue(k_, i0, i1, r0, r1, xh_b, d_b, sem):
            blk = wid + k_ * NW

            @pl.when(blk < NBLK)
            def _():
                base = blk * B
                pltpu.sync_copy(ei0_hbm.at[pl.ds(base, B)], i0)
                pltpu.sync_copy(ei1_hbm.at[pl.ds(base, B)], i1)
                if has_x:
                    pltpu.sync_copy(d_hbm.at[pl.ds(base, B)], d_b)
                    pltpu.async_copy(xh_hbm.at[pl.ds(base, B)], xh_b, sem)
                pltpu.async_copy(t_hbm.at[i0], r0, sem)
                pltpu.async_copy(t_hbm.at[i1], r1, sem)

        def finish(k_, i0, i1, r0, r1, xh_b, d_b, sem):
            blk = wid + k_ * NW

            @pl.when(blk < NBLK)
            def _():
                base = blk * B
                pltpu.make_async_copy(t_hbm.at[i0], r0, sem).wait()
                pltpu.make_async_copy(t_hbm.at[i1], r1, sem).wait()
                if has_x:
                    pltpu.make_async_copy(xh_hbm.at[pl.ds(base, B)], xh_b,
                                          sem).wait()

                if mode in ("scale", "sum"):
                    @pl.loop(0, B)
                    def _(r):
                        for cc in range(0, DF, 16):
                            c = pl.ds(cc, 16)
                            s = r0[r, c] + r1[r, c]
                            r0[r, c] = s * 0.5 if mode == "scale" else s

                    pltpu.sync_copy(r0, g_hbm.at[pl.ds(base, B)])
                elif mode == "ea":
                    @pl.loop(0, B, step=16)
                    def _(rr):
                        dvec = d_b[pl.ds(rr, 16)]
                        for j in range(16):
                            r = rr + j
                            dv = jnp.full((16,), dvec[j], jnp.float32)
                            for cc in range(0, DF, 16):
                                c = pl.ds(cc, 16)
                                v = (r0[r, c] + r1[r, c] + xh_b[r, c]) * dv
                                r0[r, c] = jnp.maximum(v + bbuf[c], 0.0)

                    pltpu.sync_copy(r0, g_hbm.at[pl.ds(base, B)])
                else:
                    lane = lax.iota(jnp.int32, 16)

                    @pl.loop(0, B, step=16)
                    def _(rr):
                        dvec = d_b[pl.ds(rr, 16)]
                        pvec = jnp.zeros((16,), jnp.float32)
                        for j in range(16):
                            r = rr + j
                            dv = jnp.full((16,), dvec[j], jnp.float32)
                            acc = jnp.zeros((16,), jnp.float32)
                            for cc in range(0, DF, 16):
                                c = pl.ds(cc, 16)
                                v = (r0[r, c] + r1[r, c] + xh_b[r, c]) * dv
                                ea_c = jnp.maximum(v + bbuf[c], 0.0)
                                acc = acc + ea_c * wmbuf[c]
                            s_j = jnp.full((16,), jnp.sum(acc), jnp.float32)
                            pvec = jnp.where(lane == j, s_j, pvec)
                        pbuf[pl.ds(rr, 16)] = pvec

                    pltpu.sync_copy(pbuf, g_hbm.at[blk])

        issue(0, i0a, i1a, r0a, r1a, xha if has_x else None,
              da if has_x else None, sema)

        @pl.loop(0, KMAX, step=2)
        def _(k_):
            issue(k_ + 1, i0b, i1b, r0b, r1b, xhb if has_x else None,
                  db if has_x else None, semb)
            finish(k_, i0a, i1a, r0a, r1a, xha if has_x else None,
                   da if has_x else None, sema)
            issue(k_ + 2, i0a, i1a, r0a, r1a, xha if has_x else None,
                  da if has_x else None, sema)
            finish(k_ + 1, i0b, i1b, r0b, r1b, xhb if has_x else None,
                   db if has_x else None, semb)

    @functools.partial(
        pl.kernel,
        mesh=plsc.VectorSubcoreMesh(**_MESH),
        compiler_params=_SC_CP,
        out_type=out_ty,
        scratch_types=scratch,
    )
    def k(*refs):
        body(refs)

    if mode in ("scale", "sum"):
        return k(tbl, ei0, ei1)
    if mode == "ea":
        return k(tbl, ei0, ei1, xh, d, bias)
    return k(tbl, ei0, ei1, xh, d, bias, wm)


def _sc_scatter(vals, ei0, ei1):
    @functools.partial(
        pl.kernel,
        mesh=plsc.VectorSubcoreMesh(**_MESH),
        out_type=jax.ShapeDtypeStruct((NC, NP, DF), jnp.float32),
        scratch_types=[
            pltpu.VMEM_SHARED((NP, DF), jnp.float32),
            pltpu.VMEM((B, DF), jnp.float32),
            pltpu.VMEM((B, DF), jnp.float32),
            pltpu.VMEM((B,), jnp.int32),
            pltpu.VMEM((B,), jnp.int32),
            pltpu.VMEM((B,), jnp.int32),
            pltpu.VMEM((B,), jnp.int32),
            pltpu.SemaphoreType.DMA,
            pltpu.SemaphoreType.DMA,
        ],
    )
    def k(v_hbm, ei0_hbm, ei1_hbm, out_hbm, s_spm,
          rowsa, rowsb, i0a, i1a, i0b, i1b, sema, semb):
        cid = lax.axis_index("c")
        sid = lax.axis_index("s")
        wid = _wid()

        @pl.loop(0, B)
        def _(r):
            @pl.loop(0, DF, step=16)
            def _(cc):
                rowsa[r, pl.ds(cc, 16)] = jnp.zeros((16,), jnp.float32)

        def issue(k_, i0, i1, rows, sem):
            blk = wid + k_ * NW

            @pl.when(blk < NBLK)
            def _():
                base = blk * B
                pltpu.sync_copy(ei0_hbm.at[pl.ds(base, B)], i0)
                pltpu.sync_copy(ei1_hbm.at[pl.ds(base, B)], i1)
                pltpu.async_copy(v_hbm.at[pl.ds(base, B)], rows, sem)

        def finish(k_, i0, i1, rows, sem):
            blk = wid + k_ * NW

            @pl.when(blk < NBLK)
            def _():
                base = blk * B
                pltpu.make_async_copy(v_hbm.at[pl.ds(base, B)], rows,
                                      sem).wait()
                pltpu.sync_copy(rows, s_spm.at[i0], add=True)
                pltpu.sync_copy(rows, s_spm.at[i1], add=True)

        @pl.loop(0, 5)
        def _(z):
            pltpu.sync_copy(rowsa, s_spm.at[pl.ds(sid * 640 + z * B, B)])

        issue(0, i0a, i1a, rowsa, sema)

        plsc.subcore_barrier()

        @pl.loop(0, KMAX, step=2)
        def _(k_):
            issue(k_ + 1, i0b, i1b, rowsb, semb)
            finish(k_, i0a, i1a, rowsa, sema)
            issue(k_ + 2, i0a, i1a, rowsa, sema)
            finish(k_ + 1, i0b, i1b, rowsb, semb)

        plsc.subcore_barrier()

        @pl.loop(0, 5)
        def _(z):
            off = sid * 640 + z * B
            pltpu.sync_copy(s_spm.at[pl.ds(off, B)],
                            out_hbm.at[cid].at[pl.ds(off, B)])

    return k(vals, ei0, ei1)


_HI = jax.lax.Precision.DEFAULT
_R = 4000
_R2 = 1024


def _tc_pre(cnt2):
    def body(c_ref, cnt_ref, inv_ref):
        c = c_ref[0] + c_ref[1]
        cnt_ref[...] = c
        ge2 = c >= 2.0
        inv_ref[...] = jnp.where(ge2, 1.0 / jnp.where(ge2, c, 1.0), 0.0)

    return pl.pallas_call(
        body,
        in_specs=[pl.BlockSpec((NC, NP), lambda: (0, 0))],
        out_specs=[pl.BlockSpec((NP,), lambda: (0,)),
                   pl.BlockSpec((NP,), lambda: (0,))],
        out_shape=[jax.ShapeDtypeStruct((NP,), jnp.float32),
                   jax.ShapeDtypeStruct((NP,), jnp.float32)],
    )(cnt2)


def _tc_xh(ea, w, wm=None):
    if wm is None:
        def body(ea_ref, w_ref, xh_ref):
            xh_ref[...] = jnp.dot(ea_ref[...], w_ref[...],
                                  preferred_element_type=jnp.float32,
                                  precision=_HI)

        return pl.pallas_call(
            body,
            grid=(NE // _R,),
            in_specs=[pl.BlockSpec((_R, DF), lambda i: (i, 0)),
                      pl.BlockSpec((DF, DF), lambda i: (0, 0))],
            out_specs=[pl.BlockSpec((_R, DF), lambda i: (i, 0))],
            out_shape=[jax.ShapeDtypeStruct((NE, DF), jnp.float32)],
        )(ea, w)[0]

    def body(ea_ref, w_ref, wm_ref, xh_ref, p_ref):
        ea_ = ea_ref[...]
        xh_ref[...] = jnp.dot(ea_, w_ref[...],
                              preferred_element_type=jnp.float32, precision=_HI)
        p_ref[...] = jnp.dot(ea_, wm_ref[...],
                             preferred_element_type=jnp.float32, precision=_HI)

    return pl.pallas_call(
        body,
        grid=(NE // _R,),
        in_specs=[pl.BlockSpec((_R, DF), lambda i: (i, 0)),
                  pl.BlockSpec((DF, DF), lambda i: (0, 0)),
                  pl.BlockSpec((DF, 1), lambda i: (0, 0))],
        out_specs=[pl.BlockSpec((_R, DF), lambda i: (i, 0)),
                   pl.BlockSpec((_R, 1), lambda i: (i, 0))],
        out_shape=[jax.ShapeDtypeStruct((NE, DF), jnp.float32),
                   jax.ShapeDtypeStruct((NE, 1), jnp.float32)],
    )(ea, w, wm)


def _tc_et(s2, inv1, w):
    def body(s_ref, inv_ref, w_ref, et_ref):
        sn = (s_ref[0] + s_ref[1]) * inv_ref[...]
        et_ref[...] = jnp.dot(sn, w_ref[...],
                              preferred_element_type=jnp.float32, precision=_HI)

    return pl.pallas_call(
        body,
        grid=(NP // _R2,),
        in_specs=[pl.BlockSpec((NC, _R2, DF), lambda i: (0, i, 0)),
                  pl.BlockSpec((_R2, 1), lambda i: (i, 0)),
                  pl.BlockSpec((DF, DF), lambda i: (0, 0))],
        out_specs=[pl.BlockSpec((_R2, DF), lambda i: (i, 0))],
        out_shape=[jax.ShapeDtypeStruct((NP, DF), jnp.float32)],
    )(s2, inv1, w)[0]


def _tc_ea(g, xh, d2, brow):
    def body(g_ref, xh_ref, d_ref, b_ref, ea_ref):
        v = (g_ref[...] + xh_ref[...]) * d_ref[...] + b_ref[...]
        ea_ref[...] = jnp.maximum(v, 0.0)

    return pl.pallas_call(
        body,
        grid=(NE // _R,),
        in_specs=[pl.BlockSpec((_R, DF), lambda i: (i, 0)),
                  pl.BlockSpec((_R, DF), lambda i: (i, 0)),
                  pl.BlockSpec((_R, 1), lambda i: (i, 0)),
                  pl.BlockSpec((1, DF), lambda i: (0, 0))],
        out_specs=[pl.BlockSpec((_R, DF), lambda i: (i, 0))],
        out_shape=[jax.ShapeDtypeStruct((NE, DF), jnp.float32)],
    )(g, xh, d2, brow)[0]


def _tc_soft(p0, p1, p2, bm, batch, eit):
    def body(p0_ref, p1_ref, p2_ref, bm_ref, b_ref, e_ref, o_ref):
        p = p0_ref[...] + p1_ref[...] + p2_ref[...] + bm_ref[0, 0]
        bt = b_ref[...]
        ei = e_ref[...]
        eb = jnp.zeros(ei.shape, jnp.int32)
        for g in range(1, NG):
            t_g = jnp.sum((bt < g).astype(jnp.int32))
            eb = eb + (ei >= t_g).astype(jnp.int32)
        msel = jnp.zeros(p.shape, jnp.float32)
        for g in range(NG):
            mg = jnp.max(jnp.where(eb == g, p, -jnp.inf))
            mg = jnp.where(jnp.isfinite(mg), mg, 0.0)
            msel = jnp.where(eb == g, mg, msel)
        ex = jnp.exp(p - msel)
        dsel = jnp.zeros(p.shape, jnp.float32)
        for g in range(NG):
            sg = jnp.sum(jnp.where(eb == g, ex, 0.0))
            dsel = jnp.where(eb == g, sg, dsel)
        o_ref[...] = ex / (dsel + 1e-16)

    nr = NE // DF
    return pl.pallas_call(
        body,
        in_specs=[pl.BlockSpec((nr, DF), lambda: (0, 0)),
                  pl.BlockSpec((nr, DF), lambda: (0, 0)),
                  pl.BlockSpec((nr, DF), lambda: (0, 0)),
                  pl.BlockSpec((1, 1), lambda: (0, 0)),
                  pl.BlockSpec((NN,), lambda: (0,)),
                  pl.BlockSpec((nr, DF), lambda: (0, 0))],
        out_specs=[pl.BlockSpec((nr, DF), lambda: (0, 0))],
        out_shape=[jax.ShapeDtypeStruct((nr, DF), jnp.float32)],
    )(p0, p1, p2, bm, batch, eit)[0]


def kernel(x, edge_index, edge_attr, batch, W0, b0, W1, b1, W2, b2,
           W_mlp, b_mlp):
    del edge_attr
    ei0 = edge_index[0]
    ei1 = edge_index[1]

    cnt2 = _sc_count(ei0, ei1)
    cnt, inv = _tc_pre(cnt2)
    inv1 = inv.reshape(NP, 1)
    d1 = _sc_edge_deg(cnt, ei0, ei1)
    d2 = d1.reshape(NE, 1)

    x_dep = lax.optimization_barrier((x, d1))[0]
    ea = _sc_gather_fused(x_dep, ei0, ei1, "scale")

    ws = (W0, W1, W2)
    bs = (b0, b1, b2)
    wms = (W_mlp[0:DF, 0], W_mlp[DF:2 * DF, 0], W_mlp[2 * DF:3 * DF, 0])

    ps = []
    for li in range(3):
        s2 = _sc_scatter(ea, ei0, ei1)
        if li == 0:
            xh = _tc_xh(ea, ws[0])
        else:
            xh, p_prev = _tc_xh(ea, ws[li], wms[li - 1].reshape(DF, 1))
            ps.append(p_prev.reshape(NE // DF, DF))
        s2b = lax.optimization_barrier((s2, xh))[0]
        et = _tc_et(s2b, inv1, ws[li])
        if li < 2:
            g = _sc_gather_fused(et, ei0, ei1, "sum")
            ea = _tc_ea(g, xh, d2, bs[li].reshape(1, DF))
        else:
            p_l = _sc_gather_fused(et, ei0, ei1, "last",
                                   xh=xh, d=d1, bias=bs[li], wm=wms[li])
            ps.append(p_l)

    out = _tc_soft(ps[0], ps[1], ps[2], b_mlp.reshape(1, 1), batch,
                   ei0.reshape(NE // DF, DF))
    return out.reshape(NE, 1)


# speedup vs baseline: 1.2327x; 1.0341x over previous
import dataclasses
import functools

import jax
import jax.numpy as jnp
from jax import lax
from jax.experimental import pallas as pl
from jax.experimental.pallas import tpu as pltpu
from jax.experimental.pallas import tpu_sc as plsc

NN = 10000
NP = 10240
NE = 160000
DF = 128
NG = 8

NC, NS = 2, 16
NW = NC * NS
B = 128
NBLK = NE // B
KMAX = -(-NBLK // NW)

_MESH = dict(core_axis_name="c", subcore_axis_name="s")

_SC_CP = pltpu.CompilerParams()
if "needs_layout_passes" in pltpu.CompilerParams.__dataclass_fields__:
    _SC_CP = dataclasses.replace(_SC_CP, needs_layout_passes=False)


def _wid():
    return lax.axis_index("s") * NC + lax.axis_index("c")


def _sc_count(ei0, ei1):
    @functools.partial(
        pl.kernel,
        mesh=plsc.VectorSubcoreMesh(**_MESH),
        out_type=jax.ShapeDtypeStruct((NC, NP), jnp.float32),
        scratch_types=[
            pltpu.VMEM_SHARED((NP,), jnp.float32),
            pltpu.VMEM((640,), jnp.float32),
            pltpu.VMEM((B,), jnp.float32),
            pltpu.VMEM((B,), jnp.int32),
            pltpu.VMEM((B,), jnp.int32),
        ],
    )
    def k(ei0_hbm, ei1_hbm, out_hbm, cnt_spm, zbuf, ones, i0, i1):
        cid = lax.axis_index("c")
        sid = lax.axis_index("s")
        wid = _wid()

        @pl.loop(0, 640, step=16)
        def _(i):
            zbuf[pl.ds(i, 16)] = jnp.zeros((16,), jnp.float32)

        @pl.loop(0, B, step=16)
        def _(i):
            ones[pl.ds(i, 16)] = jnp.full((16,), 1.0, jnp.float32)

        pltpu.sync_copy(zbuf, cnt_spm.at[pl.ds(sid * 640, 640)])

        plsc.subcore_barrier()

        @pl.loop(0, KMAX)
        def _(k_):
            blk = wid + k_ * NW

            @pl.when(blk < NBLK)
            def _():
                base = blk * B
                pltpu.sync_copy(ei0_hbm.at[pl.ds(base, B)], i0)
                pltpu.sync_copy(ei1_hbm.at[pl.ds(base, B)], i1)
                pltpu.sync_copy(ones, cnt_spm.at[i0], add=True)
                pltpu.sync_copy(ones, cnt_spm.at[i1], add=True)

        plsc.subcore_barrier()

        pltpu.sync_copy(cnt_spm.at[pl.ds(sid * 640, 640)],
                        out_hbm.at[cid].at[pl.ds(sid * 640, 640)])

    return k(ei0, ei1)


def _sc_gather_fused(tbl, ei0, ei1, mode, xh=None, d=None, bias=None, wm=None):
    has_x = mode in ("ea", "last")
    if mode == "last":
        out_ty = jax.ShapeDtypeStruct((NBLK, B), jnp.float32)
    elif mode == "scale":
        out_ty = (jax.ShapeDtypeStruct((NE, DF), jnp.float32),
                  jax.ShapeDtypeStruct((NE,), jnp.float32))
    else:
        out_ty = jax.ShapeDtypeStruct((NE, DF), jnp.float32)
    scratch = [
        pltpu.VMEM((B, DF), jnp.float32),
        pltpu.VMEM((B, DF), jnp.float32),
        pltpu.VMEM((B, DF), jnp.float32),
        pltpu.VMEM((B, DF), jnp.float32),
        pltpu.VMEM((B,), jnp.int32),
        pltpu.VMEM((B,), jnp.int32),
        pltpu.VMEM((B,), jnp.int32),
        pltpu.VMEM((B,), jnp.int32),
        pltpu.SemaphoreType.DMA,
        pltpu.SemaphoreType.DMA,
    ]
    if has_x:
        scratch += [
            pltpu.VMEM((B, DF), jnp.float32),
            pltpu.VMEM((B, DF), jnp.float32),
            pltpu.VMEM((B,), jnp.float32),
            pltpu.VMEM((B,), jnp.float32),
            pltpu.VMEM((DF,), jnp.float32),
        ]
    if mode == "last":
        scratch += [
            pltpu.VMEM((DF,), jnp.float32),
            pltpu.VMEM((B,), jnp.float32),
        ]
    if mode == "scale":
        scratch += [
            pltpu.VMEM((NP,), jnp.float32),
            pltpu.VMEM((B,), jnp.float32),
        ]

    def body(refs):
        if mode == "scale":
            (t_hbm, ei0_hbm, ei1_hbm, cnt_hbm, g_hbm, d_hbm,
             r0a, r1a, r0b, r1b, i0a, i1a, i0b, i1b, sema, semb,
             cntv, dv) = refs
        elif mode == "sum":
            (t_hbm, ei0_hbm, ei1_hbm, g_hbm,
             r0a, r1a, r0b, r1b, i0a, i1a, i0b, i1b, sema, semb) = refs
        elif mode == "ea":
            (t_hbm, ei0_hbm, ei1_hbm, xh_hbm, d_hbm, b_hbm, g_hbm,
             r0a, r1a, r0b, r1b, i0a, i1a, i0b, i1b, sema, semb,
             xha, xhb, da, db, bbuf) = refs
        else:
            (t_hbm, ei0_hbm, ei1_hbm, xh_hbm, d_hbm, b_hbm, wm_hbm, g_hbm,
             r0a, r1a, r0b, r1b, i0a, i1a, i0b, i1b, sema, semb,
             xha, xhb, da, db, bbuf, wmbuf, pbuf) = refs
        wid = _wid()
        if has_x:
            pltpu.sync_copy(b_hbm, bbuf)
        if mode == "last":
            pltpu.sync_copy(wm_hbm, wmbuf)
        if mode == "scale":
            pltpu.sync_copy(cnt_hbm, cntv)

        def issue(k_, i0, i1, r0, r1, xh_b, d_b, sem):
            blk = wid + k_ * NW

            @pl.when(blk < NBLK)
            def _():
                base = blk * B
                pltpu.sync_copy(ei0_hbm.at[pl.ds(base, B)], i0)
                pltpu.sync_copy(ei1_hbm.at[pl.ds(base, B)], i1)
                if has_x:
                    pltpu.sync_copy(d_hbm.at[pl.ds(base, B)], d_b)
                    pltpu.async_copy(xh_hbm.at[pl.ds(base, B)], xh_b, sem)
                pltpu.async_copy(t_hbm.at[i0], r0, sem)
                pltpu.async_copy(t_hbm.at[i1], r1, sem)

        def finish(k_, i0, i1, r0, r1, xh_b, d_b, sem):
            blk = wid + k_ * NW

            @pl.when(blk < NBLK)
            def _():
                base = blk * B
                pltpu.make_async_copy(t_hbm.at[i0], r0, sem).wait()
                pltpu.make_async_copy(t_hbm.at[i1], r1, sem).wait()
                if has_x:
                    pltpu.make_async_copy(xh_hbm.at[pl.ds(base, B)], xh_b,
                                          sem).wait()

                if mode in ("scale", "sum"):
                    if mode == "scale":
                        @pl.loop(0, B, step=16)
                        def _(j):
                            one = jnp.full((16,), 1.0, jnp.float32)
                            zero = jnp.zeros((16,), jnp.float32)
                            c0 = plsc.load_gather(cntv, [i0[pl.ds(j, 16)]])
                            c1 = plsc.load_gather(cntv, [i1[pl.ds(j, 16)]])
                            m0 = jnp.where(c0 != one, one, zero)
                            m1 = jnp.where(c1 != one, one, zero)
                            dv[pl.ds(j, 16)] = one / (one + m0 + m1)

                    @pl.loop(0, B)
                    def _(r):
                        for cc in range(0, DF, 16):
                            c = pl.ds(cc, 16)
                            s = r0[r, c] + r1[r, c]
                            r0[r, c] = s * 0.5 if mode == "scale" else s

                    pltpu.sync_copy(r0, g_hbm.at[pl.ds(base, B)])
                    if mode == "scale":
                        pltpu.sync_copy(dv, d_hbm.at[pl.ds(base, B)])
                elif mode == "ea":
                    @pl.loop(0, B, step=16)
                    def _(rr):
                        dvec = d_b[pl.ds(rr, 16)]
                        for j in range(16):
                            r = rr + j
                            dv = jnp.full((16,), dvec[j], jnp.float32)
                            for cc in range(0, DF, 16):
                                c = pl.ds(cc, 16)
                                v = (r0[r, c] + r1[r, c] + xh_b[r, c]) * dv
                                r0[r, c] = jnp.maximum(v + bbuf[c], 0.0)

                    pltpu.sync_copy(r0, g_hbm.at[pl.ds(base, B)])
                else:
                    lane = lax.iota(jnp.int32, 16)

                    @pl.loop(0, B, step=16)
                    def _(rr):
                        dvec = d_b[pl.ds(rr, 16)]
                        pvec = jnp.zeros((16,), jnp.float32)
                        for j in range(16):
                            r = rr + j
                            dv = jnp.full((16,), dvec[j], jnp.float32)
                            acc = jnp.zeros((16,), jnp.float32)
                            for cc in range(0, DF, 16):
                                c = pl.ds(cc, 16)
                                v = (r0[r, c] + r1[r, c] + xh_b[r, c]) * dv
                                ea_c = jnp.maximum(v + bbuf[c], 0.0)
                                acc = acc + ea_c * wmbuf[c]
                            s_j = jnp.full((16,), jnp.sum(acc), jnp.float32)
                            pvec = jnp.where(lane == j, s_j, pvec)
                        pbuf[pl.ds(rr, 16)] = pvec

                    pltpu.sync_copy(pbuf, g_hbm.at[blk])

        issue(0, i0a, i1a, r0a, r1a, xha if has_x else None,
              da if has_x else None, sema)

        @pl.loop(0, KMAX, step=2)
        def _(k_):
            issue(k_ + 1, i0b, i1b, r0b, r1b, xhb if has_x else None,
                  db if has_x else None, semb)
            finish(k_, i0a, i1a, r0a, r1a, xha if has_x else None,
                   da if has_x else None, sema)
            issue(k_ + 2, i0a, i1a, r0a, r1a, xha if has_x else None,
                  da if has_x else None, sema)
            finish(k_ + 1, i0b, i1b, r0b, r1b, xhb if has_x else None,
                   db if has_x else None, semb)

    @functools.partial(
        pl.kernel,
        mesh=plsc.VectorSubcoreMesh(**_MESH),
        compiler_params=_SC_CP,
        out_type=out_ty,
        scratch_types=scratch,
    )
    def k(*refs):
        body(refs)

    if mode == "scale":
        return k(tbl, ei0, ei1, d)
    if mode == "sum":
        return k(tbl, ei0, ei1)
    if mode == "ea":
        return k(tbl, ei0, ei1, xh, d, bias)
    return k(tbl, ei0, ei1, xh, d, bias, wm)


def _sc_scatter(vals, ei0, ei1):
    @functools.partial(
        pl.kernel,
        mesh=plsc.VectorSubcoreMesh(**_MESH),
        out_type=jax.ShapeDtypeStruct((NC, NP, DF), jnp.float32),
        scratch_types=[
            pltpu.VMEM_SHARED((NP, DF), jnp.float32),
            pltpu.VMEM((B, DF), jnp.float32),
            pltpu.VMEM((B, DF), jnp.float32),
            pltpu.VMEM((B,), jnp.int32),
            pltpu.VMEM((B,), jnp.int32),
            pltpu.VMEM((B,), jnp.int32),
            pltpu.VMEM((B,), jnp.int32),
            pltpu.SemaphoreType.DMA,
            pltpu.SemaphoreType.DMA,
        ],
    )
    def k(v_hbm, ei0_hbm, ei1_hbm, out_hbm, s_spm,
          rowsa, rowsb, i0a, i1a, i0b, i1b, sema, semb):
        cid = lax.axis_index("c")
        sid = lax.axis_index("s")
        wid = _wid()

        @pl.loop(0, B)
        def _(r):
            @pl.loop(0, DF, step=16)
            def _(cc):
                rowsa[r, pl.ds(cc, 16)] = jnp.zeros((16,), jnp.float32)

        def issue(k_, i0, i1, rows, sem):
            blk = wid + k_ * NW

            @pl.when(blk < NBLK)
            def _():
                base = blk * B
                pltpu.sync_copy(ei0_hbm.at[pl.ds(base, B)], i0)
                pltpu.sync_copy(ei1_hbm.at[pl.ds(base, B)], i1)
                pltpu.async_copy(v_hbm.at[pl.ds(base, B)], rows, sem)

        def finish(k_, i0, i1, rows, sem):
            blk = wid + k_ * NW

            @pl.when(blk < NBLK)
            def _():
                base = blk * B
                pltpu.make_async_copy(v_hbm.at[pl.ds(base, B)], rows,
                                      sem).wait()
                pltpu.sync_copy(rows, s_spm.at[i0], add=True)
                pltpu.sync_copy(rows, s_spm.at[i1], add=True)

        @pl.loop(0, 5)
        def _(z):
            pltpu.sync_copy(rowsa, s_spm.at[pl.ds(sid * 640 + z * B, B)])

        issue(0, i0a, i1a, rowsa, sema)

        plsc.subcore_barrier()

        @pl.loop(0, KMAX, step=2)
        def _(k_):
            issue(k_ + 1, i0b, i1b, rowsb, semb)
            finish(k_, i0a, i1a, rowsa, sema)
            issue(k_ + 2, i0a, i1a, rowsa, sema)
            finish(k_ + 1, i0b, i1b, rowsb, semb)

        plsc.subcore_barrier()

        @pl.loop(0, 5)
        def _(z):
            off = sid * 640 + z * B
            pltpu.sync_copy(s_spm.at[pl.ds(off, B)],
                            out_hbm.at[cid].at[pl.ds(off, B)])

    return k(vals, ei0, ei1)


_HI = jax.lax.Precision.DEFAULT
_R = 4000
_R2 = 1024


def _tc_pre(cnt2):
    def body(c_ref, cnt_ref, inv_ref):
        c = c_ref[0] + c_ref[1]
        cnt_ref[...] = c
        ge2 = c >= 2.0
        inv_ref[...] = jnp.where(ge2, 1.0 / jnp.where(ge2, c, 1.0), 0.0)

    return pl.pallas_call(
        body,
        in_specs=[pl.BlockSpec((NC, NP), lambda: (0, 0))],
        out_specs=[pl.BlockSpec((NP,), lambda: (0,)),
                   pl.BlockSpec((NP,), lambda: (0,))],
        out_shape=[jax.ShapeDtypeStruct((NP,), jnp.float32),
                   jax.ShapeDtypeStruct((NP,), jnp.float32)],
    )(cnt2)


def _tc_xh(ea, w, wm=None):
    if wm is None:
        def body(ea_ref, w_ref, xh_ref):
            xh_ref[...] = jnp.dot(ea_ref[...], w_ref[...],
                                  preferred_element_type=jnp.float32,
                                  precision=_HI)

        return pl.pallas_call(
            body,
            grid=(NE // _R,),
            in_specs=[pl.BlockSpec((_R, DF), lambda i: (i, 0)),
                      pl.BlockSpec((DF, DF), lambda i: (0, 0))],
            out_specs=[pl.BlockSpec((_R, DF), lambda i: (i, 0))],
            out_shape=[jax.ShapeDtypeStruct((NE, DF), jnp.float32)],
        )(ea, w)[0]

    def body(ea_ref, w_ref, wm_ref, xh_ref, p_ref):
        ea_ = ea_ref[...]
        xh_ref[...] = jnp.dot(ea_, w_ref[...],
                              preferred_element_type=jnp.float32, precision=_HI)
        p_ref[...] = jnp.dot(ea_, wm_ref[...],
                             preferred_element_type=jnp.float32, precision=_HI)

    return pl.pallas_call(
        body,
        grid=(NE // _R,),
        in_specs=[pl.BlockSpec((_R, DF), lambda i: (i, 0)),
                  pl.BlockSpec((DF, DF), lambda i: (0, 0)),
                  pl.BlockSpec((DF, 1), lambda i: (0, 0))],
        out_specs=[pl.BlockSpec((_R, DF), lambda i: (i, 0)),
                   pl.BlockSpec((_R, 1), lambda i: (i, 0))],
        out_shape=[jax.ShapeDtypeStruct((NE, DF), jnp.float32),
                   jax.ShapeDtypeStruct((NE, 1), jnp.float32)],
    )(ea, w, wm)


def _tc_et(s2, inv1, w):
    def body(s_ref, inv_ref, w_ref, et_ref):
        sn = (s_ref[0] + s_ref[1]) * inv_ref[...]
        et_ref[...] = jnp.dot(sn, w_ref[...],
                              preferred_element_type=jnp.float32, precision=_HI)

    return pl.pallas_call(
        body,
        grid=(NP // _R2,),
        in_specs=[pl.BlockSpec((NC, _R2, DF), lambda i: (0, i, 0)),
                  pl.BlockSpec((_R2, 1), lambda i: (i, 0)),
                  pl.BlockSpec((DF, DF), lambda i: (0, 0))],
        out_specs=[pl.BlockSpec((_R2, DF), lambda i: (i, 0))],
        out_shape=[jax.ShapeDtypeStruct((NP, DF), jnp.float32)],
    )(s2, inv1, w)[0]


def _tc_ea(g, xh, d2, brow):
    def body(g_ref, xh_ref, d_ref, b_ref, ea_ref):
        v = (g_ref[...] + xh_ref[...]) * d_ref[...] + b_ref[...]
        ea_ref[...] = jnp.maximum(v, 0.0)

    return pl.pallas_call(
        body,
        grid=(NE // _R,),
        in_specs=[pl.BlockSpec((_R, DF), lambda i: (i, 0)),
                  pl.BlockSpec((_R, DF), lambda i: (i, 0)),
                  pl.BlockSpec((_R, 1), lambda i: (i, 0)),
                  pl.BlockSpec((1, DF), lambda i: (0, 0))],
        out_specs=[pl.BlockSpec((_R, DF), lambda i: (i, 0))],
        out_shape=[jax.ShapeDtypeStruct((NE, DF), jnp.float32)],
    )(g, xh, d2, brow)[0]


def _tc_soft(p0, p1, p2, bm, batch, eit):
    def body(p0_ref, p1_ref, p2_ref, bm_ref, b_ref, e_ref, o_ref):
        p = p0_ref[...] + p1_ref[...] + p2_ref[...] + bm_ref[0, 0]
        bt = b_ref[...]
        ei = e_ref[...]
        eb = jnp.zeros(ei.shape, jnp.int32)
        for g in range(1, NG):
            t_g = jnp.sum((bt < g).astype(jnp.int32))
            eb = eb + (ei >= t_g).astype(jnp.int32)
        msel = jnp.zeros(p.shape, jnp.float32)
        for g in range(NG):
            mg = jnp.max(jnp.where(eb == g, p, -jnp.inf))
            mg = jnp.where(jnp.isfinite(mg), mg, 0.0)
            msel = jnp.where(eb == g, mg, msel)
        ex = jnp.exp(p - msel)
        dsel = jnp.zeros(p.shape, jnp.float32)
        for g in range(NG):
            sg = jnp.sum(jnp.where(eb == g, ex, 0.0))
            dsel = jnp.where(eb == g, sg, dsel)
        o_ref[...] = ex / (dsel + 1e-16)

    nr = NE // DF
    return pl.pallas_call(
        body,
        in_specs=[pl.BlockSpec((nr, DF), lambda: (0, 0)),
                  pl.BlockSpec((nr, DF), lambda: (0, 0)),
                  pl.BlockSpec((nr, DF), lambda: (0, 0)),
                  pl.BlockSpec((1, 1), lambda: (0, 0)),
                  pl.BlockSpec((NN,), lambda: (0,)),
                  pl.BlockSpec((nr, DF), lambda: (0, 0))],
        out_specs=[pl.BlockSpec((nr, DF), lambda: (0, 0))],
        out_shape=[jax.ShapeDtypeStruct((nr, DF), jnp.float32)],
    )(p0, p1, p2, bm, batch, eit)[0]


def kernel(x, edge_index, edge_attr, batch, W0, b0, W1, b1, W2, b2,
           W_mlp, b_mlp):
    del edge_attr
    ei0 = edge_index[0]
    ei1 = edge_index[1]

    cnt2 = _sc_count(ei0, ei1)
    cnt, inv = _tc_pre(cnt2)
    inv1 = inv.reshape(NP, 1)

    ea, d1 = _sc_gather_fused(x, ei0, ei1, "scale", d=cnt)
    d2 = d1.reshape(NE, 1)

    ws = (W0, W1, W2)
    bs = (b0, b1, b2)
    wms = (W_mlp[0:DF, 0], W_mlp[DF:2 * DF, 0], W_mlp[2 * DF:3 * DF, 0])

    ps = []
    for li in range(3):
        s2 = _sc_scatter(ea, ei0, ei1)
        if li == 0:
            xh = _tc_xh(ea, ws[0])
        else:
            xh, p_prev = _tc_xh(ea, ws[li], wms[li - 1].reshape(DF, 1))
            ps.append(p_prev.reshape(NE // DF, DF))
        s2b = lax.optimization_barrier((s2, xh))[0]
        et = _tc_et(s2b, inv1, ws[li])
        if li < 2:
            g = _sc_gather_fused(et, ei0, ei1, "sum")
            ea = _tc_ea(g, xh, d2, bs[li].reshape(1, DF))
        else:
            p_l = _sc_gather_fused(et, ei0, ei1, "last",
                                   xh=xh, d=d1, bias=bs[li], wm=wms[li])
            ps.append(p_l)

    out = _tc_soft(ps[0], ps[1], ps[2], b_mlp.reshape(1, 1), batch,
                   ei0.reshape(NE // DF, DF))
    return out.reshape(NE, 1)
